# baseline (speedup 1.0000x reference)
import jax
import jax.numpy as jnp
from jax import lax
from jax.experimental import pallas as pl
from jax.experimental.pallas import tpu as pltpu
from jax.experimental.pallas import tpu_sc as plsc

VOCAB = 1000000
EMB = 16
NIG = 5
NGG = 21
OUT = 10
BATCH = 16384

NC, NS, L = 2, 16, 16
NW = NC * NS
BPW = BATCH // NW
CH = 128
NCHUNK = BPW // CH
NG = BPW // L

ROWS_PER_BLK = 8
NBLK = VOCAB // ROWS_PER_BLK
BLK = ROWS_PER_BLK * EMB

WCOLS = EMB + NIG + NGG
WSTRIDE = WCOLS + 1
WFLAT = OUT * WSTRIDE
WPAD = 512

NSPLAT = OUT * (EMB + 1)
SPLAT_ROWS = (NSPLAT * L + 127) // 128
SPLAT_ROWS_PAD = 24

OUT_ROWS_W = BPW * OUT // 128
OUT_ROWS = BATCH * OUT // 128


def _body(blk_hbm, off_hbm, ig_hbm, gg_hbm, table_hbm, w_hbm, wsplat_hbm,
          out_hbm, blk_v, off_v, ig_v, gg_v, rows_v, w_v, wsplat_v, out_v,
          sem):
    wid = lax.axis_index("s") * NC + lax.axis_index("c")

    pltpu.sync_copy(blk_hbm.at[wid], blk_v)
    pltpu.sync_copy(off_hbm.at[wid], off_v)
    pltpu.sync_copy(ig_hbm.at[wid], ig_v)
    pltpu.sync_copy(gg_hbm.at[wid], gg_v)
    pltpu.sync_copy(w_hbm, w_v)
    pltpu.sync_copy(wsplat_hbm, wsplat_v)

    copies = [
        pltpu.async_copy(table_hbm.at[blk_v.at[c]],
                         rows_v.at[pl.ds(c * CH, CH)], sem)
        for c in range(NCHUNK)
    ]

    iota = lax.iota(jnp.int32, L)
    iota10 = iota * OUT

    for cp in copies:
        cp.wait()

    def _wsplat(j, k):
        flat = (j * (EMB + 1) + k) * L
        return wsplat_v[flat // 128, pl.ds(flat % 128, L)]

    def g_body(g, carry):
        gbase = g * L
        row_ids = gbase + iota
        offv = off_v[pl.ds(gbase, L)]
        es = [
            jnp.maximum(
                plsc.load_gather(rows_v, [row_ids, offv + k]), 0.0)
            for k in range(EMB)
        ]
        igv = ig_v[pl.ds(gbase, L)]
        ggv = gg_v[pl.ds(gbase, L)]
        outbase = gbase * OUT + iota10
        for j in range(OUT):
            base = j * WSTRIDE
            acc = _wsplat(j, EMB)
            for k in range(EMB):
                acc = acc + es[k] * _wsplat(j, k)
            acc = acc + plsc.load_gather(w_v, [(base + EMB) + igv])
            acc = acc + plsc.load_gather(w_v, [(base + EMB + NIG) + ggv])
            flat = outbase + j
            plsc.store_scatter(out_v, [flat >> 7, flat & 127], acc)
        return carry

    lax.fori_loop(0, NG, g_body, 0)

    pltpu.sync_copy(out_v, out_hbm.at[pl.ds(wid * OUT_ROWS_W, OUT_ROWS_W)])


@jax.jit
def _run(blk3, off2, ig2, gg2, table_r, wflat, wsplat):
    mesh = plsc.VectorSubcoreMesh(core_axis_name="c", subcore_axis_name="s",
                                  num_cores=NC, num_subcores=NS)
    return pl.kernel(
        _body,
        out_type=jax.ShapeDtypeStruct((OUT_ROWS, 128), jnp.float32),
        mesh=mesh,
        compiler_params=pltpu.CompilerParams(needs_layout_passes=False),
        scratch_types=[
            pltpu.VMEM((NCHUNK, CH), jnp.int32),
            pltpu.VMEM((BPW,), jnp.int32),
            pltpu.VMEM((BPW,), jnp.int32),
            pltpu.VMEM((BPW,), jnp.int32),
            pltpu.VMEM((BPW, BLK), jnp.float32),
            pltpu.VMEM((WPAD,), jnp.float32),
            pltpu.VMEM((SPLAT_ROWS_PAD, 128), jnp.float32),
            pltpu.VMEM((OUT_ROWS_W, 128), jnp.float32),
            pltpu.SemaphoreType.DMA,
        ],
    )(blk3, off2, ig2, gg2, table_r, wflat, wsplat)


def kernel(item_indices, index_group_indices, garment_group_indices,
           emb_table, W, b):
    idxi = item_indices.astype(jnp.int32)
    blk3 = (idxi // ROWS_PER_BLK).reshape(NW, NCHUNK, CH)
    off2 = ((idxi % ROWS_PER_BLK) * EMB).reshape(NW, BPW)
    ig2 = index_group_indices.astype(jnp.int32).reshape(NW, BPW)
    gg2 = garment_group_indices.astype(jnp.int32).reshape(NW, BPW)
    table_r = emb_table.reshape(NBLK, BLK)
    wb = jnp.concatenate([W, b[:, None]], axis=1)
    wflat = jnp.pad(wb.reshape(WFLAT), (0, WPAD - WFLAT))
    wsplat = jnp.broadcast_to(
        jnp.concatenate([W[:, :EMB], b[:, None]], axis=1)[:, :, None],
        (OUT, EMB + 1, L)).reshape(NSPLAT * L)
    wsplat = jnp.pad(wsplat, (0, SPLAT_ROWS_PAD * 128 - NSPLAT * L))
    wsplat = wsplat.reshape(SPLAT_ROWS_PAD, 128).astype(jnp.float32)
    out_flat = _run(blk3, off2, ig2, gg2, table_r, wflat, wsplat)
    return out_flat.reshape(BATCH, OUT)


# speedup vs baseline: 4.8613x; 4.8613x over previous
import jax
import jax.numpy as jnp
from jax import lax
from jax.experimental import pallas as pl
from jax.experimental.pallas import tpu as pltpu
from jax.experimental.pallas import tpu_sc as plsc

VOCAB = 1000000
EMB = 16
NIG = 5
NGG = 21
OUT = 10
BATCH = 16384

NC, NS, L = 2, 16, 16
NW = NC * NS
BPW = BATCH // NW
NG = BPW // L

NIDX = BPW * EMB
NCH = NIDX // 128

WCOLS = EMB + NIG + NGG
WSTRIDE = WCOLS + 1
WFLAT = OUT * WSTRIDE
WPAD = 512

NSPLAT = OUT * (EMB + 1)
SPLAT_ROWS_PAD = 24

OUT_ROWS_W = BPW * OUT // 128
OUT_ROWS = BATCH * OUT // 128

VALID_COLS = 999936
TAIL = VOCAB - VALID_COLS
TILES_PER_STRIPE = 7812
STRIPE_WORDS = TILES_PER_STRIPE * 1024
MAIN_WORDS = 2 * STRIPE_WORDS
FLAT_ROWS = MAIN_WORDS // 1024 + 1
RL_TPC = 61
RL_NCH = 8


def _relayout_body(table_hbm, tail_hbm, flat_hbm, chunk_a, chunk_b,
                   sem_r, sem_w):
    wid = lax.axis_index("s") * NC + lax.axis_index("c")
    s = wid // 16
    j = wid % 16
    r0 = pl.multiple_of(s * 8, 8)
    t0 = j * 488 + jnp.minimum(j, 4)

    bufs = (chunk_a, chunk_b)

    def read_tiles(buf, tbase, ntiles):
        return [
            pltpu.async_copy(
                table_hbm.at[pl.ds(r0, 8),
                             pl.ds(pl.multiple_of((tbase + t) * 128, 128),
                                   128)],
                buf.at[t], sem_r)
            for t in range(ntiles)
        ]

    def write_tiles(buf, tbase, ntiles):
        return pltpu.async_copy(
            buf.at[pl.ds(0, ntiles)],
            flat_hbm.at[pl.ds(s * TILES_PER_STRIPE + tbase, ntiles)],
            sem_w)

    writes = [None] * (RL_NCH + 1)
    for ch in range(RL_NCH):
        if ch >= 2:
            writes[ch - 2].wait()
        rs = read_tiles(bufs[ch % 2], t0 + ch * RL_TPC, RL_TPC)
        for r in rs:
            r.wait()
        writes[ch] = write_tiles(bufs[ch % 2], t0 + ch * RL_TPC, RL_TPC)
    writes[RL_NCH - 2].wait()

    @pl.when(j < 4)
    def _extra():
        tb = t0 + RL_NCH * RL_TPC
        read_tiles(bufs[RL_NCH % 2], tb, 1)[0].wait()
        write_tiles(bufs[RL_NCH % 2], tb, 1).wait()

    writes[RL_NCH - 1].wait()

    @pl.when(wid == 0)
    def _tail():
        pltpu.async_copy(tail_hbm, chunk_a.at[0], sem_r).wait()
        pltpu.async_copy(chunk_a.at[pl.ds(0, 1)],
                         flat_hbm.at[pl.ds(FLAT_ROWS - 1, 1)], sem_w).wait()


def _body(idx_hbm, ig_hbm, gg_hbm, table_hbm, w_hbm, wsplat_hbm,
          out_hbm, idxl_v, ig_v, gg_v, rows_v, w_v, wsplat_v, out_v, sem):
    wid = lax.axis_index("s") * NC + lax.axis_index("c")

    pltpu.sync_copy(idx_hbm.at[wid], idxl_v)
    pltpu.sync_copy(ig_hbm.at[wid], ig_v)
    pltpu.sync_copy(gg_hbm.at[wid], gg_v)
    pltpu.sync_copy(w_hbm, w_v)
    pltpu.sync_copy(wsplat_hbm, wsplat_v)

    copies = [
        pltpu.async_copy(table_hbm.at[idxl_v.at[c]], rows_v.at[c], sem)
        for c in range(NCH)
    ]

    for cp in copies:
        cp.wait()

    def _wsplat(j, k):
        flat = (j * (EMB + 1) + k) * L
        return wsplat_v[flat // 128, pl.ds(flat % 128, L)]

    def g_body(g, carry):
        gbase = g * L
        es = [
            jnp.maximum(
                rows_v[2 * g + (k // 8), pl.ds((k % 8) * L, L)], 0.0)
            for k in range(EMB)
        ]
        igv = ig_v[pl.ds(gbase, L)]
        ggv = gg_v[pl.ds(gbase, L)]
        for j in range(OUT):
            base = j * WSTRIDE
            acc = _wsplat(j, EMB)
            for k in range(EMB):
                acc = acc + es[k] * _wsplat(j, k)
            acc = acc + plsc.load_gather(w_v, [(base + EMB) + igv])
            acc = acc + plsc.load_gather(w_v, [(base + EMB + NIG) + ggv])
            out_v[j, pl.ds(gbase, L)] = acc
        return carry

    lax.fori_loop(0, NG, g_body, 0)

    pltpu.sync_copy(out_v, out_hbm.at[:, pl.ds(wid * BPW, BPW)])


@jax.jit
def _run(idx3, ig2, gg2, table2d, tail2d, wflat, wsplat):
    mesh = plsc.VectorSubcoreMesh(core_axis_name="c", subcore_axis_name="s",
                                  num_cores=NC, num_subcores=NS)
    table_tiles = pl.kernel(
        _relayout_body,
        out_type=jax.ShapeDtypeStruct((FLAT_ROWS, 8, 128), jnp.float32),
        mesh=mesh,
        compiler_params=pltpu.CompilerParams(needs_layout_passes=False),
        scratch_types=[
            pltpu.VMEM((RL_TPC, 8, 128), jnp.float32),
            pltpu.VMEM((RL_TPC, 8, 128), jnp.float32),
            pltpu.SemaphoreType.DMA,
            pltpu.SemaphoreType.DMA,
        ],
    )(table2d, tail2d)
    table_flat = table_tiles.reshape(FLAT_ROWS * 1024)
    return pl.kernel(
        _body,
        out_type=jax.ShapeDtypeStruct((OUT, BATCH), jnp.float32),
        mesh=mesh,
        compiler_params=pltpu.CompilerParams(needs_layout_passes=False),
        scratch_types=[
            pltpu.VMEM((NCH, 128), jnp.int32),
            pltpu.VMEM((BPW,), jnp.int32),
            pltpu.VMEM((BPW,), jnp.int32),
            pltpu.VMEM((NCH, 128), jnp.float32),
            pltpu.VMEM((WPAD,), jnp.float32),
            pltpu.VMEM((SPLAT_ROWS_PAD, 128), jnp.float32),
            pltpu.VMEM((OUT, BPW), jnp.float32),
            pltpu.SemaphoreType.DMA,
        ],
    )(idx3, ig2, gg2, table_flat, wflat, wsplat)


def kernel(item_indices, index_group_indices, garment_group_indices,
           emb_table, W, b):
    idxi = item_indices.astype(jnp.int32)
    cols = idxi.reshape(NW, NG, 1, L)
    kk = jnp.arange(EMB, dtype=jnp.int32).reshape(1, 1, EMB, 1)
    main = ((kk // 8) * TILES_PER_STRIPE + cols // 128) * 1024 \
        + (kk % 8) * 128 + (cols % 128)
    tailw = MAIN_WORDS + (cols - VALID_COLS) * EMB + kk
    idx3 = jnp.where(cols < VALID_COLS, main, tailw).reshape(NW, NCH, 128)
    ig2 = index_group_indices.astype(jnp.int32).reshape(NW, BPW)
    gg2 = garment_group_indices.astype(jnp.int32).reshape(NW, BPW)
    table2d = emb_table.T
    tail2d = emb_table[VALID_COLS:, :].reshape(8, 128)
    wb = jnp.concatenate([W, b[:, None]], axis=1)
    wflat = jnp.pad(wb.reshape(WFLAT), (0, WPAD - WFLAT))
    wsplat = jnp.broadcast_to(
        jnp.concatenate([W[:, :EMB], b[:, None]], axis=1)[:, :, None],
        (OUT, EMB + 1, L)).reshape(NSPLAT * L)
    wsplat = jnp.pad(wsplat, (0, SPLAT_ROWS_PAD * 128 - NSPLAT * L))
    wsplat = wsplat.reshape(SPLAT_ROWS_PAD, 128).astype(jnp.float32)
    out_cm = _run(idx3, ig2, gg2, table2d, tail2d, wflat, wsplat)
    return out_cm.T


# speedup vs baseline: 4.8754x; 1.0029x over previous
import jax
import jax.numpy as jnp
from jax import lax
from jax.experimental import pallas as pl
from jax.experimental.pallas import tpu as pltpu
from jax.experimental.pallas import tpu_sc as plsc

VOCAB = 1000000
EMB = 16
NIG = 5
NGG = 21
OUT = 10
BATCH = 16384

NC, NS, L = 2, 16, 16
NW = NC * NS
BPW = BATCH // NW
NG = BPW // L

NIDX = BPW * EMB
NCH = NIDX // 128

WCOLS = EMB + NIG + NGG
WSTRIDE = WCOLS + 1
WFLAT = OUT * WSTRIDE
WPAD = 512

NSPLAT = OUT * (EMB + 1)
SPLAT_ROWS_PAD = 24

OUT_ROWS_W = BPW * OUT // 128
OUT_ROWS = BATCH * OUT // 128

VALID_COLS = 999936
TAIL = VOCAB - VALID_COLS
TILES_PER_STRIPE = 7812
STRIPE_WORDS = TILES_PER_STRIPE * 1024
MAIN_WORDS = 2 * STRIPE_WORDS
FLAT_ROWS = MAIN_WORDS // 1024 + 1
RL_TPC = 61
RL_NCH = 8


def _relayout_body(table_hbm, tail_hbm, flat_hbm, chunk_a, chunk_b,
                   sem_r, sem_w):
    wid = lax.axis_index("s") * NC + lax.axis_index("c")
    s = wid // 16
    j = wid % 16
    r0 = pl.multiple_of(s * 8, 8)
    t0 = j * 488 + jnp.minimum(j, 4)

    bufs = (chunk_a, chunk_b)

    def read_tiles(buf, tbase, ntiles):
        return [
            pltpu.async_copy(
                table_hbm.at[pl.ds(r0, 8),
                             pl.ds(pl.multiple_of((tbase + t) * 128, 128),
                                   128)],
                buf.at[t], sem_r)
            for t in range(ntiles)
        ]

    def write_tiles(buf, tbase, ntiles):
        return pltpu.async_copy(
            buf.at[pl.ds(0, ntiles)],
            flat_hbm.at[pl.ds(s * TILES_PER_STRIPE + tbase, ntiles)],
            sem_w)

    writes = [None] * (RL_NCH + 1)
    for ch in range(RL_NCH):
        if ch >= 2:
            writes[ch - 2].wait()
        rs = read_tiles(bufs[ch % 2], t0 + ch * RL_TPC, RL_TPC)
        for r in rs:
            r.wait()
        writes[ch] = write_tiles(bufs[ch % 2], t0 + ch * RL_TPC, RL_TPC)
    writes[RL_NCH - 2].wait()

    @pl.when(j < 4)
    def _extra():
        tb = t0 + RL_NCH * RL_TPC
        read_tiles(bufs[RL_NCH % 2], tb, 1)[0].wait()
        write_tiles(bufs[RL_NCH % 2], tb, 1).wait()

    writes[RL_NCH - 1].wait()

    @pl.when(wid == 0)
    def _tail():
        pltpu.async_copy(tail_hbm, chunk_a.at[0], sem_r).wait()
        pltpu.async_copy(chunk_a.at[pl.ds(0, 1)],
                         flat_hbm.at[pl.ds(FLAT_ROWS - 1, 1)], sem_w).wait()


def _body(idx_hbm, ig_hbm, gg_hbm, table_hbm, w_hbm, wsplat_hbm,
          out_hbm, idxl_v, ig_v, gg_v, rows_v, w_v, wsplat_v, out_v,
          sem, sem_g):
    wid = lax.axis_index("s") * NC + lax.axis_index("c")

    pltpu.sync_copy(idx_hbm.at[wid], idxl_v)
    stages = [
        pltpu.async_copy(ig_hbm.at[wid], ig_v, sem),
        pltpu.async_copy(gg_hbm.at[wid], gg_v, sem),
        pltpu.async_copy(w_hbm, w_v, sem),
        pltpu.async_copy(wsplat_hbm, wsplat_v, sem),
    ]

    copies = [
        pltpu.async_copy(table_hbm.at[idxl_v.at[c]], rows_v.at[c], sem_g)
        for c in range(NCH)
    ]
    for st in stages:
        st.wait()

    def _wsplat(j, k):
        flat = (j * (EMB + 1) + k) * L
        return wsplat_v[flat // 128, pl.ds(flat % 128, L)]

    def g_body(g, carry):
        gbase = g * L
        es = [
            jnp.maximum(
                rows_v[2 * g + (k // 8), pl.ds((k % 8) * L, L)], 0.0)
            for k in range(EMB)
        ]
        igv = ig_v[pl.ds(gbase, L)]
        ggv = gg_v[pl.ds(gbase, L)]
        for j in range(OUT):
            base = j * WSTRIDE
            acc = _wsplat(j, EMB)
            for k in range(EMB):
                acc = acc + es[k] * _wsplat(j, k)
            acc = acc + plsc.load_gather(w_v, [(base + EMB) + igv])
            acc = acc + plsc.load_gather(w_v, [(base + EMB + NIG) + ggv])
            out_v[j, pl.ds(gbase, L)] = acc
        return carry

    for cp in copies[:NCH // 2]:
        cp.wait()
    lax.fori_loop(0, NG // 2, g_body, 0)
    for cp in copies[NCH // 2:]:
        cp.wait()
    lax.fori_loop(NG // 2, NG, g_body, 0)

    pltpu.sync_copy(out_v, out_hbm.at[:, pl.ds(wid * BPW, BPW)])


@jax.jit
def _run(idx3, ig2, gg2, table2d, tail2d, wflat, wsplat):
    mesh = plsc.VectorSubcoreMesh(core_axis_name="c", subcore_axis_name="s",
                                  num_cores=NC, num_subcores=NS)
    table_tiles = pl.kernel(
        _relayout_body,
        out_type=jax.ShapeDtypeStruct((FLAT_ROWS, 8, 128), jnp.float32),
        mesh=mesh,
        compiler_params=pltpu.CompilerParams(needs_layout_passes=False),
        scratch_types=[
            pltpu.VMEM((RL_TPC, 8, 128), jnp.float32),
            pltpu.VMEM((RL_TPC, 8, 128), jnp.float32),
            pltpu.SemaphoreType.DMA,
            pltpu.SemaphoreType.DMA,
        ],
    )(table2d, tail2d)
    table_flat = table_tiles.reshape(FLAT_ROWS * 1024)
    return pl.kernel(
        _body,
        out_type=jax.ShapeDtypeStruct((OUT, BATCH), jnp.float32),
        mesh=mesh,
        compiler_params=pltpu.CompilerParams(needs_layout_passes=False),
        scratch_types=[
            pltpu.VMEM((NCH, 128), jnp.int32),
            pltpu.VMEM((BPW,), jnp.int32),
            pltpu.VMEM((BPW,), jnp.int32),
            pltpu.VMEM((NCH, 128), jnp.float32),
            pltpu.VMEM((WPAD,), jnp.float32),
            pltpu.VMEM((SPLAT_ROWS_PAD, 128), jnp.float32),
            pltpu.VMEM((OUT, BPW), jnp.float32),
            pltpu.SemaphoreType.DMA,
            pltpu.SemaphoreType.DMA,
        ],
    )(idx3, ig2, gg2, table_flat, wflat, wsplat)


def kernel(item_indices, index_group_indices, garment_group_indices,
           emb_table, W, b):
    idxi = item_indices.astype(jnp.int32)
    cols = idxi.reshape(NW, NG, 1, L)
    kk = jnp.arange(EMB, dtype=jnp.int32).reshape(1, 1, EMB, 1)
    main = ((kk // 8) * TILES_PER_STRIPE + cols // 128) * 1024 \
        + (kk % 8) * 128 + (cols % 128)
    tailw = MAIN_WORDS + (cols - VALID_COLS) * EMB + kk
    idx3 = jnp.where(cols < VALID_COLS, main, tailw).reshape(NW, NCH, 128)
    ig2 = index_group_indices.astype(jnp.int32).reshape(NW, BPW)
    gg2 = garment_group_indices.astype(jnp.int32).reshape(NW, BPW)
    table2d = emb_table.T
    tail2d = emb_table[VALID_COLS:, :].reshape(8, 128)
    wb = jnp.concatenate([W, b[:, None]], axis=1)
    wflat = jnp.pad(wb.reshape(WFLAT), (0, WPAD - WFLAT))
    wsplat = jnp.broadcast_to(
        jnp.concatenate([W[:, :EMB], b[:, None]], axis=1)[:, :, None],
        (OUT, EMB + 1, L)).reshape(NSPLAT * L)
    wsplat = jnp.pad(wsplat, (0, SPLAT_ROWS_PAD * 128 - NSPLAT * L))
    wsplat = wsplat.reshape(SPLAT_ROWS_PAD, 128).astype(jnp.float32)
    out_cm = _run(idx3, ig2, gg2, table2d, tail2d, wflat, wsplat)
    return out_cm.T
